# baseline (speedup 1.0000x reference)
import functools

import jax
import jax.numpy as jnp
from jax.experimental import pallas as pl
from jax.experimental.pallas import tpu as pltpu


def _conv_full(x_ref, wt_ref, b_ref, H, W, KH):
    X = x_ref[0].astype(jnp.bfloat16)
    Cin, P = X.shape
    q = KH // 2
    lane = jax.lax.broadcasted_iota(jnp.int32, (Cin, P), 1)
    wpos = lane % W
    acc = None
    for kh in range(KH):
        dh = kh - q
        for kw in range(KH):
            dw = kw - q
            delta = dh * W + dw
            if delta > 0:
                xs = jnp.concatenate(
                    [X[:, delta:], jnp.zeros((Cin, delta), X.dtype)], axis=1)
            elif delta < 0:
                xs = jnp.concatenate(
                    [jnp.zeros((Cin, -delta), X.dtype), X[:, :delta]], axis=1)
            else:
                xs = X
            if dw > 0:
                xs = jnp.where(wpos >= W - dw, jnp.bfloat16(0.0), xs)
            elif dw < 0:
                xs = jnp.where(wpos < -dw, jnp.bfloat16(0.0), xs)
            d = jax.lax.dot_general(
                xs, wt_ref[kh, kw],
                dimension_numbers=(((0,), (0,)), ((), ())),
                preferred_element_type=jnp.float32)
            acc = d if acc is None else acc + d
    return acc + b_ref[...]


def _stats_kernel(H, W, KH, x_ref, wt_ref, b_ref, st_ref):
    y = _conv_full(x_ref, wt_ref, b_ref, H, W, KH)
    s = jnp.sum(y, axis=0, keepdims=True)
    ss = jnp.sum(y * y, axis=0, keepdims=True)
    st_ref[0] = jnp.concatenate([s, ss], axis=0)


def _bn_pool_relu_kernel(H, W, KH, x_ref, wt_ref, b_ref, sc_ref, sh_ref, o_ref):
    y = _conv_full(x_ref, wt_ref, b_ref, H, W, KH)
    Cout = y.shape[1]
    n = y * sc_ref[...] + sh_ref[...]
    n4 = n.reshape(H // 2, 2, W // 2, 2, Cout)
    pooled = jnp.maximum(jnp.maximum(n4[:, 0, :, 0], n4[:, 0, :, 1]),
                         jnp.maximum(n4[:, 1, :, 0], n4[:, 1, :, 1]))
    res = jnp.maximum(pooled.reshape(H * W // 4, Cout), 0.0)
    o_ref[0] = jnp.transpose(res, (1, 0))


def kernel(x_nchw, w_oihw, bias, gamma, beta):
    eps = 1e-5
    N, Cin, H, W = x_nchw.shape
    Cout, Cin2, KH, KW = w_oihw.shape
    assert Cin2 == Cin and KH == KW and KH % 2 == 1
    assert H % 2 == 0 and W % 2 == 0
    Hh, Wh = H // 2, W // 2
    nrows = Hh * Wh

    x3 = x_nchw.reshape(N, Cin, H * W)
    Wt = jnp.transpose(w_oihw, (2, 3, 1, 0)).astype(jnp.bfloat16)
    b2 = bias.astype(jnp.float32).reshape(1, Cout)

    cparams = pltpu.CompilerParams(
        dimension_semantics=("parallel",),
        vmem_limit_bytes=64 * 1024 * 1024)

    x_spec = pl.BlockSpec((1, Cin, H * W), lambda n: (n, 0, 0))
    wt_spec = pl.BlockSpec((KH, KW, Cin, Cout), lambda n: (0, 0, 0, 0))
    vec_spec = pl.BlockSpec((1, Cout), lambda n: (0, 0))

    st = pl.pallas_call(
        functools.partial(_stats_kernel, H, W, KH),
        out_shape=jax.ShapeDtypeStruct((N, 2, Cout), jnp.float32),
        grid=(N,),
        in_specs=[x_spec, wt_spec, vec_spec],
        out_specs=pl.BlockSpec((1, 2, Cout), lambda n: (n, 0, 0)),
        compiler_params=cparams,
    )(x3, Wt, b2)

    stats = jnp.sum(st, axis=0)
    count = jnp.float32(N * H * W)
    mean = stats[0] / count
    var = jnp.maximum(stats[1] / count - mean * mean, 0.0)
    inv = jax.lax.rsqrt(var + eps)
    g = gamma.astype(jnp.float32)
    scale = (inv * g).reshape(1, Cout)
    shift = (beta.astype(jnp.float32) - mean * inv * g).reshape(1, Cout)

    out = pl.pallas_call(
        functools.partial(_bn_pool_relu_kernel, H, W, KH),
        out_shape=jax.ShapeDtypeStruct((N, Cout, nrows), jnp.float32),
        grid=(N,),
        in_specs=[x_spec, wt_spec, vec_spec, vec_spec, vec_spec],
        out_specs=pl.BlockSpec((1, Cout, nrows), lambda n: (n, 0, 0)),
        compiler_params=cparams,
    )(x3, Wt, b2, scale, shift)

    return out.reshape(N, Cout, Hh, Wh)


# speedup vs baseline: 1.0141x; 1.0141x over previous
import functools

import jax
import jax.numpy as jnp
from jax.experimental import pallas as pl
from jax.experimental.pallas import tpu as pltpu


def _conv_full(x_ref, wt_ref, H, W, KH):
    X = x_ref[0].astype(jnp.bfloat16)
    Cin, P = X.shape
    q = KH // 2
    lane = jax.lax.broadcasted_iota(jnp.int32, (Cin, P), 1)
    wpos = lane % W
    acc = None
    for kh in range(KH):
        dh = kh - q
        for kw in range(KH):
            dw = kw - q
            delta = dh * W + dw
            if delta > 0:
                xs = jnp.concatenate(
                    [X[:, delta:], jnp.zeros((Cin, delta), X.dtype)], axis=1)
            elif delta < 0:
                xs = jnp.concatenate(
                    [jnp.zeros((Cin, -delta), X.dtype), X[:, :delta]], axis=1)
            else:
                xs = X
            if dw > 0:
                xs = jnp.where(wpos >= W - dw, jnp.bfloat16(0.0), xs)
            elif dw < 0:
                xs = jnp.where(wpos < -dw, jnp.bfloat16(0.0), xs)
            d = jax.lax.dot_general(
                xs, wt_ref[kh, kw],
                dimension_numbers=(((0,), (0,)), ((), ())),
                preferred_element_type=jnp.float32)
            acc = d if acc is None else acc + d
    return acc


def _stats_kernel(H, W, KH, x_ref, wt_ref, st_ref):
    y = _conv_full(x_ref, wt_ref, H, W, KH)
    s = jnp.sum(y, axis=0, keepdims=True)
    ss = jnp.sum(y * y, axis=0, keepdims=True)
    st_ref[0] = jnp.concatenate([s, ss], axis=0)


def _bn_pool_relu_kernel(H, W, KH, x_ref, wt_ref, sc_ref, sh_ref, o_ref):
    y = _conv_full(x_ref, wt_ref, H, W, KH)
    Cout = y.shape[1]
    n = y * sc_ref[...] + sh_ref[...]
    n4 = n.reshape(H // 2, 2, W // 2, 2, Cout)
    pooled = jnp.maximum(jnp.maximum(n4[:, 0, :, 0], n4[:, 0, :, 1]),
                         jnp.maximum(n4[:, 1, :, 0], n4[:, 1, :, 1]))
    res = jnp.maximum(pooled.reshape(H * W // 4, Cout), 0.0)
    o_ref[0] = jnp.transpose(res, (1, 0))


def kernel(x_nchw, w_oihw, bias, gamma, beta):
    eps = 1e-5
    N, Cin, H, W = x_nchw.shape
    Cout, Cin2, KH, KW = w_oihw.shape
    assert Cin2 == Cin and KH == KW and KH % 2 == 1
    assert H % 2 == 0 and W % 2 == 0
    Hh, Wh = H // 2, W // 2
    nrows = Hh * Wh

    x3 = x_nchw.reshape(N, Cin, H * W)
    Wt = jnp.transpose(w_oihw, (2, 3, 1, 0)).astype(jnp.bfloat16)

    cparams = pltpu.CompilerParams(
        dimension_semantics=("parallel",),
        vmem_limit_bytes=64 * 1024 * 1024)

    x_spec = pl.BlockSpec((1, Cin, H * W), lambda n: (n, 0, 0))
    wt_spec = pl.BlockSpec((KH, KW, Cin, Cout), lambda n: (0, 0, 0, 0))
    vec_spec = pl.BlockSpec((1, Cout), lambda n: (0, 0))

    st = pl.pallas_call(
        functools.partial(_stats_kernel, H, W, KH),
        out_shape=jax.ShapeDtypeStruct((N, 2, Cout), jnp.float32),
        grid=(N,),
        in_specs=[x_spec, wt_spec],
        out_specs=pl.BlockSpec((1, 2, Cout), lambda n: (n, 0, 0)),
        compiler_params=cparams,
    )(x3, Wt)

    stats = jnp.sum(st, axis=0)
    count = jnp.float32(N * H * W)
    mean_raw = stats[0] / count
    var = jnp.maximum(stats[1] / count - mean_raw * mean_raw, 0.0)
    inv = jax.lax.rsqrt(var + eps)
    g = gamma.astype(jnp.float32)
    scale = (inv * g).reshape(1, Cout)
    shift = (beta.astype(jnp.float32) - mean_raw * inv * g).reshape(1, Cout)

    out = pl.pallas_call(
        functools.partial(_bn_pool_relu_kernel, H, W, KH),
        out_shape=jax.ShapeDtypeStruct((N, Cout, nrows), jnp.float32),
        grid=(N,),
        in_specs=[x_spec, wt_spec, vec_spec, vec_spec],
        out_specs=pl.BlockSpec((1, Cout, nrows), lambda n: (n, 0, 0)),
        compiler_params=cparams,
    )(x3, Wt, scale, shift)

    return out.reshape(N, Cout, Hh, Wh)
